# Initial kernel scaffold; baseline (speedup 1.0000x reference)
import jax
import jax.numpy as jnp
from jax.experimental import pallas as pl


def kernel(x, edge_index, edge_attr, params):
    raise NotImplementedError("write your pallas kernel here")


# speedup vs baseline: 17.8023x; 17.8023x over previous
import functools

import jax
import jax.numpy as jnp
from jax import lax
from jax.experimental import pallas as pl
from jax.experimental.pallas import tpu as pltpu
from jax.experimental.pallas import tpu_sc as plsc

N = 10000
E = 320000
D = 128
ED = 4
H = 8
CH = 16
EPS = 1e-5
RS = 1.0 / (1.0 + EPS) ** 0.5

NC = 2
NS = 16
HH = H // NC
DW = D // NC
ROW = 128
EPT = E // NS
CK = 80
NCH = EPT // CK
RA = 640
RS0 = 624

_f32 = jnp.float32
_i32 = jnp.int32


@functools.cache
def _build_gat_sc(with_ea: bool):
  mesh = plsc.VectorSubcoreMesh(core_axis_name="c", subcore_axis_name="s",
                                num_cores=NC, num_subcores=NS)
  scratch = [
      pltpu.VMEM((CK,), _i32),
      pltpu.VMEM((CK,), _i32),
      pltpu.VMEM((CK,), _i32),
      pltpu.VMEM((CK, D), _f32),
      pltpu.VMEM((CK, D), _f32),
      pltpu.VMEM((CK, ROW), _f32),
      pltpu.VMEM((CK * HH,), _f32),
      pltpu.VMEM((CK * ED,), _f32),
      pltpu.VMEM_SHARED((N, ROW), _f32),
      pltpu.SemaphoreType.DMA,
  ]

  @functools.partial(
      pl.kernel,
      out_type=jax.ShapeDtypeStruct((NC * N, ROW), _f32),
      mesh=mesh,
      scratch_types=scratch,
  )
  def gat_sc(d_hbm, gs_hbm, gd_hbm, ae_hbm, xw_hbm, eaf_hbm,
             out_hbm, dv, gsv, gdv, rin, rbv, rout, aev, eav, osh, sem):
    c = lax.axis_index("c")
    t = lax.axis_index("s")
    lane = lax.iota(_i32, 16)
    l4m = lax.bitwise_and(lane, 3)
    for e in range(CK):
      for k in range(ROW // 16):
        rout[e, pl.ds(k * 16, 16)] = jnp.zeros((16,), _f32)

    @pl.loop(0, RA // CK)
    def zero_j(j):
      pltpu.sync_copy(rout, osh.at[pl.ds(t * RS0 + j * CK, CK)])

    plsc.subcore_barrier()

    @pl.loop(0, NCH)
    def chunk(g):
      base = t * EPT + g * CK
      pltpu.sync_copy(d_hbm.at[pl.ds(base, CK)], dv)
      pltpu.sync_copy(gs_hbm.at[pl.ds(c * E + base, CK)], gsv)
      pltpu.sync_copy(gd_hbm.at[pl.ds(c * E + base, CK)], gdv)
      pltpu.sync_copy(ae_hbm.at[pl.ds(c * (E * HH) + base * HH, CK * HH)],
                      aev)
      if with_ea:
        pltpu.sync_copy(eaf_hbm.at[pl.ds(base * ED, CK * ED)], eav)
      c1 = pltpu.async_copy(xw_hbm.at[gsv], rin, sem)
      c2 = pltpu.async_copy(xw_hbm.at[gdv], rbv, sem)
      c1.wait()
      c2.wait()
      for grp in range(CK // 4):
        e0 = grp * 4
        aeg = aev[pl.ds(grp * 16, 16)]
        if with_ea:
          eag = eav[pl.ds(grp * 16, 16)]
        for i in range(4):
          e = e0 + i
          row_a = rin[e, pl.ds(DW, 16)]
          b_rep = rbv[e, pl.ds(DW, 16)].at[l4m + HH].get(
              mode="promise_in_bounds")
          ae_rep = aeg.at[l4m + 4 * i].get(mode="promise_in_bounds")
          tt = row_a + b_rep + ae_rep
          ex = jnp.exp(jnp.maximum(tt, 0.2 * tt))
          for hh in range(HH):
            bc = ex.at[lane * 0 + hh].get(mode="promise_in_bounds")
            rout[e, pl.ds(hh * 16, 16)] = rin[e, pl.ds(hh * 16, 16)] * bc
          if with_ea:
            ea_rep = eag.at[l4m + 4 * i].get(mode="promise_in_bounds")
            tail = jnp.where(
                lane < 4, ex,
                jnp.where(lane < 8, ea_rep,
                          jnp.where(lane == 8, 1.0, 0.0)))
          else:
            tail = jnp.where(lane < 4, ex, 0.0)
          rout[e, pl.ds(DW, 16)] = tail
      pltpu.sync_copy(rout, osh.at[dv], add=True)

    plsc.subcore_barrier()

    @pl.loop(0, RA // CK)
    def out_j(j):
      r0 = t * RS0 + j * CK
      pltpu.sync_copy(osh.at[pl.ds(r0, CK)],
                      out_hbm.at[pl.ds(c * N + r0, CK)])

  return gat_sc


def _gat_sc_l0(*args):
  return _build_gat_sc(True)(*args)


def _gat_sc_l1(*args):
  return _build_gat_sc(False)(*args)


_BLK = 1000


def _enc_body(x_ref, w_ref, b_ref, g_ref, bb_ref, o_ref):
  h = jnp.dot(x_ref[...], w_ref[...], preferred_element_type=_f32)
  h = (h + b_ref[...]) * (g_ref[...] * RS) + bb_ref[...]
  o_ref[...] = jnp.maximum(h, 0.0)


def _enc_call(x, w, b, g, bb):
  return pl.pallas_call(
      _enc_body,
      grid=(N // _BLK,),
      in_specs=[
          pl.BlockSpec((_BLK, D), lambda i: (i, 0)),
          pl.BlockSpec((D, D), lambda i: (0, 0)),
          pl.BlockSpec((1, D), lambda i: (0, 0)),
          pl.BlockSpec((1, D), lambda i: (0, 0)),
          pl.BlockSpec((1, D), lambda i: (0, 0)),
      ],
      out_specs=pl.BlockSpec((_BLK, D), lambda i: (i, 0)),
      out_shape=jax.ShapeDtypeStruct((N, D), _f32),
  )(x, w, b.reshape(1, D), g.reshape(1, D), bb.reshape(1, D))


def _xw_body(h_ref, w_ref, asf_ref, adf_ref, xw_ref, att_ref):
  xw = jnp.dot(h_ref[...], w_ref[...], preferred_element_type=_f32)
  xw_ref[...] = xw
  a_s = jnp.sum((xw * asf_ref[...]).reshape(_BLK, H, CH), axis=-1)
  a_d = jnp.sum((xw * adf_ref[...]).reshape(_BLK, H, CH), axis=-1)
  att_ref[...] = jnp.concatenate(
      [a_s, a_d, jnp.zeros((_BLK, D - 2 * H), _f32)], axis=1)


def _xw_call(h, w, asf, adf):
  return pl.pallas_call(
      _xw_body,
      grid=(N // _BLK,),
      in_specs=[
          pl.BlockSpec((_BLK, D), lambda i: (i, 0)),
          pl.BlockSpec((D, D), lambda i: (0, 0)),
          pl.BlockSpec((1, D), lambda i: (0, 0)),
          pl.BlockSpec((1, D), lambda i: (0, 0)),
      ],
      out_specs=[
          pl.BlockSpec((_BLK, D), lambda i: (i, 0)),
          pl.BlockSpec((_BLK, D), lambda i: (i, 0)),
      ],
      out_shape=[
          jax.ShapeDtypeStruct((N, D), _f32),
          jax.ShapeDtypeStruct((N, D), _f32),
      ],
  )(h, w, asf.reshape(1, D), adf.reshape(1, D))


def _aedge_body(ea_ref, m_ref, o_ref):
  o_ref[...] = jnp.dot(ea_ref[...], m_ref[...], preferred_element_type=_f32)


def _aedge_call(ea_r, mt):
  return pl.pallas_call(
      _aedge_body,
      grid=(N // _BLK,),
      in_specs=[
          pl.BlockSpec((_BLK, D), lambda i: (i, 0)),
          pl.BlockSpec((D, 2 * 256), lambda i: (0, 0)),
      ],
      out_specs=pl.BlockSpec((_BLK, 2 * 256), lambda i: (i, 0)),
      out_shape=jax.ShapeDtypeStruct((N, 2 * 256), _f32),
  )(ea_r, mt)


def _post_body(msg_ref, aux_ref, att_ref, xw_ref, hp_ref, m_ref, b_ref,
               g_ref, bb_ref, o_ref):
  aux = aux_ref[...]
  exs = aux[:, :H]
  la = aux[:, H:H + ED] / jnp.maximum(aux[:, H + ED:H + ED + 1], 1.0)
  att = att_ref[...]
  a_self = (att[:, :H] + att[:, H:2 * H]
            + jnp.dot(la, m_ref[...][:ED, :H], preferred_element_type=_f32))
  a_self = jnp.maximum(a_self, 0.2 * a_self)
  ex_self = jnp.exp(a_self)
  inv = 1.0 / (exs + ex_self + 1e-16)

  def rep16(a):
    return jnp.broadcast_to(a[:, :, None], (_BLK, H, CH)).reshape(_BLK, D)

  out = (msg_ref[...] + xw_ref[...] * rep16(ex_self)) * rep16(inv)
  hh = (out + b_ref[...]) * (g_ref[...] * RS) + bb_ref[...]
  o_ref[...] = jnp.maximum(hh, 0.0) + 0.5 * hp_ref[...]


def _post_call(msg, aux, att, xw, hp, m_pad, b, g, bb):
  return pl.pallas_call(
      _post_body,
      grid=(N // _BLK,),
      in_specs=[
          pl.BlockSpec((_BLK, D), lambda i: (i, 0)),
          pl.BlockSpec((_BLK, 16), lambda i: (i, 0)),
          pl.BlockSpec((_BLK, D), lambda i: (i, 0)),
          pl.BlockSpec((_BLK, D), lambda i: (i, 0)),
          pl.BlockSpec((_BLK, D), lambda i: (i, 0)),
          pl.BlockSpec((8, D), lambda i: (0, 0)),
          pl.BlockSpec((1, D), lambda i: (0, 0)),
          pl.BlockSpec((1, D), lambda i: (0, 0)),
          pl.BlockSpec((1, D), lambda i: (0, 0)),
      ],
      out_specs=pl.BlockSpec((_BLK, D), lambda i: (i, 0)),
      out_shape=jax.ShapeDtypeStruct((N, D), _f32),
  )(msg, aux, att, xw, hp, m_pad, b.reshape(1, D), g.reshape(1, D),
    bb.reshape(1, D))


def _head_body(h_ref, wr1, br1, wr2, br2, wr3, br3, wv1, bv1, wv2, bv2,
               wt1, bt1, wt2, bt2, o_ref):
  h = h_ref[...]
  hm = jnp.mean(h, axis=0, keepdims=True)
  hx = jnp.max(h, axis=0, keepdims=True)
  hg = jnp.concatenate([hm, hx], axis=1)

  def mm(a, w):
    return jnp.dot(a, w[...], preferred_element_type=_f32)

  r = jnp.maximum(mm(hg, wr1) + br1[...], 0.0)
  r = jnp.maximum(mm(r, wr2) + br2[...], 0.0)
  risk = mm(r, wr3) + br3[...]
  v = jnp.maximum(mm(hg, wv1) + bv1[...], 0.0)
  v = mm(v, wv2) + bv2[...]
  vol = jnp.log1p(jnp.exp(-jnp.abs(v))) + jnp.maximum(v, 0.0)
  t = jnp.maximum(mm(hg, wt1) + bt1[...], 0.0)
  ret = mm(t, wt2) + bt2[...]
  o_ref[...] = jnp.concatenate(
      [risk, vol, ret, jnp.zeros((5, D), _f32)], axis=0)


def _head_call(h, ws):
  full = lambda shp: pl.BlockSpec(shp, lambda: tuple(0 for _ in shp))
  specs = [full((N, D))]
  for w in ws:
    specs.append(full(w.shape))
  return pl.pallas_call(
      _head_body,
      in_specs=specs,
      out_specs=full((8, D)),
      out_shape=jax.ShapeDtypeStruct((8, D), _f32),
  )(h, *ws)


def _pad_to(a, shape):
  out = jnp.zeros(shape, _f32)
  return out.at[tuple(slice(0, s) for s in a.shape)].set(a)


def kernel(x, edge_index, edge_attr, params):
  p = params
  s = edge_index[0]
  d = edge_index[1]

  h = _enc_call(x, p["W_enc"], p["b_enc"], p["g_enc"], p["beta_enc"])

  g0, g1 = p["gat0"], p["gat1"]
  m0 = jnp.einsum("jhc,hc->jh", g0["We"].reshape(ED, H, CH), g0["att_edge"])
  m1 = jnp.einsum("jhc,hc->jh", g1["We"].reshape(ED, H, CH), g1["att_edge"])
  eye32 = jnp.eye(32, dtype=_f32)
  mt = jnp.concatenate([jnp.kron(eye32, m0), jnp.kron(eye32, m1)], axis=1)
  ea_r = edge_attr.reshape(E // 32, D)
  ae_both = _aedge_call(ea_r, mt)

  ea_flat = edge_attr.reshape(E * ED)
  gs = jnp.concatenate([s, s + N])
  gd = jnp.concatenate([d, d + N])
  sums = cnt = None

  for l, (gp, m_l, sck) in enumerate(
      [(g0, m0, _gat_sc_l0), (g1, m1, _gat_sc_l1)]):
    xw, att = _xw_call(h, gp["W"], gp["att_src"].reshape(D),
                       gp["att_dst"].reshape(D))
    zpad = jnp.zeros((N, D - DW - 2 * HH), _f32)
    xwf = jnp.concatenate([
        jnp.concatenate(
            [xw[:, :DW], att[:, 0:HH], att[:, H:H + HH], zpad], axis=1),
        jnp.concatenate(
            [xw[:, DW:], att[:, HH:H], att[:, H + HH:2 * H], zpad], axis=1),
    ], axis=0)
    ael = ae_both[:, l * 256:(l + 1) * 256].reshape(E, H)
    ae_il = ael.reshape(E // 4, 4, NC, HH).transpose(2, 0, 1, 3).reshape(-1)
    sc_out = sck(d, gs, gd, ae_il, xwf, ea_flat)
    msg = jnp.concatenate([sc_out[:N, :DW], sc_out[N:, :DW]], axis=1)
    exs = jnp.concatenate([sc_out[:N, DW:DW + HH],
                           sc_out[N:, DW:DW + HH]], axis=1)
    if l == 0:
      sums = sc_out[:N, DW + HH:DW + HH + ED]
      cnt = sc_out[:N, DW + HH + ED:DW + HH + ED + 1]
    aux = jnp.concatenate([exs, sums, cnt, jnp.zeros((N, 3), _f32)], axis=1)
    h = _post_call(msg, aux, att, xw, h, _pad_to(m_l, (8, D)),
                   gp["bias"], gp["bn_g"], gp["bn_b"])

  ws = [
      _pad_to(p["Wr1"], (256, D)), _pad_to(p["br1"].reshape(1, -1), (1, D)),
      _pad_to(p["Wr2"], (D, D)), _pad_to(p["br2"].reshape(1, -1), (1, D)),
      _pad_to(p["Wr3"], (D, D)), _pad_to(p["br3"].reshape(1, -1), (1, D)),
      _pad_to(p["Wv1"], (256, D)), _pad_to(p["bv1"].reshape(1, -1), (1, D)),
      _pad_to(p["Wv2"], (D, D)), _pad_to(p["bv2"].reshape(1, -1), (1, D)),
      _pad_to(p["Wt1"], (256, D)), _pad_to(p["bt1"].reshape(1, -1), (1, D)),
      _pad_to(p["Wt2"], (D, D)), _pad_to(p["bt2"].reshape(1, -1), (1, D)),
  ]
  ho = _head_call(h, ws)
  risk = ho[0:1, 0:1]
  vol = ho[1:2, 0:1]
  ret = ho[2:3, 0:1]
  return risk, vol, ret, h
